# baseline (speedup 1.0000x reference)
import functools

import jax
import jax.numpy as jnp
from jax import lax
from jax.experimental import pallas as pl
from jax.experimental.pallas import tpu as pltpu
from jax.experimental.pallas import tpu_sc as plsc

_B, _L, _D = 4096, 200, 64
_V = 1000
_N = _B * _L
_NW = 32
_ROWS_W = _N // _NW
_CHUNK = 256
_GRP = 16
_NCHUNK = _ROWS_W // _CHUNK
_STAGE = _CHUNK * _D


def _build():
    mesh = plsc.VectorSubcoreMesh(core_axis_name="c", subcore_axis_name="s")

    @functools.partial(
        pl.kernel,
        mesh=mesh,
        out_type=jax.ShapeDtypeStruct((_N * _D,), jnp.float32),
        compiler_params=pltpu.CompilerParams(
            use_tc_tiling_on_sc=False, needs_layout_passes=False),
        scratch_types=[
            pltpu.VMEM((_V * _D,), jnp.float32),
            pltpu.VMEM((_ROWS_W,), jnp.int32),
            pltpu.VMEM((_STAGE,), jnp.float32),
            pltpu.VMEM((_STAGE,), jnp.float32),
            pltpu.SemaphoreType.DMA,
            pltpu.SemaphoreType.DMA,
        ],
    )
    def gather_kernel(table_hbm, idx_hbm, out_hbm, table_v, idx_v,
                      stage_a, stage_b, sem_a, sem_b):
        wid = lax.axis_index("s") * 2 + lax.axis_index("c")
        rbase = wid * _ROWS_W
        pltpu.sync_copy(table_hbm, table_v)
        pltpu.sync_copy(idx_hbm.at[pl.ds(rbase, _ROWS_W)], idx_v)

        lane = lax.iota(jnp.int32, 16)
        lane64 = lane * _D

        def fill(chunk_i, stage):
            def grp(g, carry):
                rows = idx_v[pl.ds(chunk_i * _CHUNK + g * _GRP, _GRP)]
                toff = rows * _D
                soff = lane64 + g * (_GRP * _D)
                for d in range(_D):
                    v = plsc.load_gather(table_v, [toff + d])
                    plsc.store_scatter(stage, [soff + d], v)
                return carry
            lax.fori_loop(0, _CHUNK // _GRP, grp, 0)

        def out_slice(chunk_i):
            return out_hbm.at[pl.ds((rbase + chunk_i * _CHUNK) * _D, _STAGE)]

        fill(0, stage_a)
        pltpu.async_copy(stage_a, out_slice(0), sem_a)
        fill(1, stage_b)
        pltpu.async_copy(stage_b, out_slice(1), sem_b)

        def body(i, carry):
            pltpu.make_async_copy(stage_a, out_slice(2 * i), sem_a).wait()
            fill(2 * i, stage_a)
            pltpu.async_copy(stage_a, out_slice(2 * i), sem_a)
            pltpu.make_async_copy(stage_b, out_slice(2 * i + 1), sem_b).wait()
            fill(2 * i + 1, stage_b)
            pltpu.async_copy(stage_b, out_slice(2 * i + 1), sem_b)
            return carry

        lax.fori_loop(1, _NCHUNK // 2, body, 0)
        pltpu.make_async_copy(stage_a, out_slice(0), sem_a).wait()
        pltpu.make_async_copy(stage_b, out_slice(1), sem_b).wait()

    return gather_kernel


_gather = _build()


def kernel(idx, node_type_embed, degree_embed):
    idx0 = idx[:, :, 0].reshape(_N)
    out = _gather(node_type_embed.reshape(_V * _D), idx0)
    return out.reshape(_B, _L, _D)


# speedup vs baseline: 1.3957x; 1.3957x over previous
import functools

import jax
import jax.numpy as jnp
from jax import lax
from jax.experimental import pallas as pl
from jax.experimental.pallas import tpu as pltpu
from jax.experimental.pallas import tpu_sc as plsc

_B, _L, _D = 4096, 200, 64
_V = 1000
_N = _B * _L
_NW = 32
_ROWS_W = _N // _NW
_CHUNK = 256
_GRP = 16
_NCHUNK = _ROWS_W // _CHUNK
_STAGE = _CHUNK * _D


def _build():
    mesh = plsc.VectorSubcoreMesh(core_axis_name="c", subcore_axis_name="s")

    @functools.partial(
        pl.kernel,
        mesh=mesh,
        out_type=jax.ShapeDtypeStruct((_N * _D,), jnp.float32),
        compiler_params=pltpu.CompilerParams(
            use_tc_tiling_on_sc=False, needs_layout_passes=False),
        scratch_types=[
            pltpu.VMEM((_V * _D,), jnp.float32),
            pltpu.VMEM((_ROWS_W,), jnp.int32),
            pltpu.VMEM((_STAGE,), jnp.float32),
            pltpu.VMEM((_STAGE,), jnp.float32),
            pltpu.SemaphoreType.DMA,
            pltpu.SemaphoreType.DMA,
        ],
    )
    def gather_kernel(table_hbm, idx_hbm, out_hbm, table_v, idx_v,
                      stage_a, stage_b, sem_a, sem_b):
        wid = lax.axis_index("s") * 2 + lax.axis_index("c")
        rbase = wid * _ROWS_W
        pltpu.sync_copy(table_hbm, table_v)
        pltpu.sync_copy(idx_hbm.at[pl.ds(rbase, _ROWS_W)], idx_v)

        lane = lax.iota(jnp.int32, 16)
        lane64 = lane * _D

        def fill(chunk_i, stage):
            @plsc.parallel_loop(0, _CHUNK // _GRP)
            def grp(g):
                rows = idx_v[pl.ds(chunk_i * _CHUNK + g * _GRP, _GRP)]
                toff = rows * _D
                soff = lane64 + g * (_GRP * _D)
                for d in range(_D):
                    v = plsc.load_gather(table_v, [toff + d])
                    plsc.store_scatter(stage, [soff + d], v)

        def out_slice(chunk_i):
            return out_hbm.at[pl.ds((rbase + chunk_i * _CHUNK) * _D, _STAGE)]

        fill(0, stage_a)
        pltpu.async_copy(stage_a, out_slice(0), sem_a)
        fill(1, stage_b)
        pltpu.async_copy(stage_b, out_slice(1), sem_b)

        def body(i, carry):
            pltpu.make_async_copy(stage_a, out_slice(2 * i), sem_a).wait()
            fill(2 * i, stage_a)
            pltpu.async_copy(stage_a, out_slice(2 * i), sem_a)
            pltpu.make_async_copy(stage_b, out_slice(2 * i + 1), sem_b).wait()
            fill(2 * i + 1, stage_b)
            pltpu.async_copy(stage_b, out_slice(2 * i + 1), sem_b)
            return carry

        lax.fori_loop(1, _NCHUNK // 2, body, 0)
        pltpu.make_async_copy(stage_a, out_slice(0), sem_a).wait()
        pltpu.make_async_copy(stage_b, out_slice(1), sem_b).wait()

    return gather_kernel


_gather = _build()


def kernel(idx, node_type_embed, degree_embed):
    idx0 = idx[:, :, 0].reshape(_N)
    out = _gather(node_type_embed.reshape(_V * _D), idx0)
    return out.reshape(_B, _L, _D)


# speedup vs baseline: 3.6119x; 2.5878x over previous
import functools

import jax
import jax.numpy as jnp
from jax import lax
from jax.experimental import pallas as pl
from jax.experimental.pallas import tpu as pltpu
from jax.experimental.pallas import tpu_sc as plsc

_B, _L, _D = 4096, 200, 64
_V = 1000
_N = _B * _L
_NW = 32
_ROWS_W = _N // _NW
_CHUNK = 512
_SUB = 128
_KSUB = _CHUNK // _SUB
_NCHUNK = _ROWS_W // _CHUNK


def _build():
    mesh = plsc.VectorSubcoreMesh(core_axis_name="c", subcore_axis_name="s")

    @functools.partial(
        pl.kernel,
        mesh=mesh,
        out_type=jax.ShapeDtypeStruct((_N, _D), jnp.float32),
        compiler_params=pltpu.CompilerParams(use_tc_tiling_on_sc=False),
        scratch_types=[
            pltpu.VMEM((_ROWS_W,), jnp.int32),
            pltpu.VMEM((_CHUNK, _D), jnp.float32),
            pltpu.VMEM((_CHUNK, _D), jnp.float32),
            pltpu.VMEM_SHARED((_V, _D), jnp.float32),
            pltpu.SemaphoreType.DMA,
            pltpu.SemaphoreType.DMA,
            pltpu.SemaphoreType.DMA,
            pltpu.SemaphoreType.DMA,
        ],
    )
    def gather_kernel(table_hbm, idx_hbm, out_hbm, idx_v, buf_a, buf_b,
                      table_sp, gsem_a, gsem_b, ssem_a, ssem_b):
        sid = lax.axis_index("s")
        wid = sid * 2 + lax.axis_index("c")
        rbase = wid * _ROWS_W

        @pl.when(sid == 0)
        def _():
            pltpu.sync_copy(table_hbm, table_sp)

        pltpu.sync_copy(idx_hbm.at[pl.ds(rbase, _ROWS_W)], idx_v)
        plsc.subcore_barrier()

        def fire_gathers(c, buf, gsem):
            return [
                pltpu.async_copy(
                    table_sp.at[idx_v.at[pl.ds(c * _CHUNK + j * _SUB, _SUB)]],
                    buf.at[pl.ds(j * _SUB, _SUB)], gsem)
                for j in range(_KSUB)
            ]

        def fire_store(c, buf, ssem):
            pltpu.async_copy(
                buf, out_hbm.at[pl.ds(rbase + c * _CHUNK, _CHUNK)], ssem)

        def drain_store(buf, ssem):
            pltpu.make_async_copy(
                buf, out_hbm.at[pl.ds(rbase, _CHUNK)], ssem).wait()

        def pair(i2, steady):
            c0, c1 = 2 * i2, 2 * i2 + 1
            if steady:
                drain_store(buf_a, ssem_a)
            cps_a = fire_gathers(c0, buf_a, gsem_a)
            if steady:
                drain_store(buf_b, ssem_b)
            cps_b = fire_gathers(c1, buf_b, gsem_b)
            for cp in cps_a:
                cp.wait()
            fire_store(c0, buf_a, ssem_a)
            for cp in cps_b:
                cp.wait()
            fire_store(c1, buf_b, ssem_b)

        pair(0, False)

        def body(i2, carry):
            pair(i2, True)
            return carry

        lax.fori_loop(1, _NCHUNK // 2, body, 0)
        drain_store(buf_a, ssem_a)
        drain_store(buf_b, ssem_b)

    return gather_kernel


_gather = _build()


def kernel(idx, node_type_embed, degree_embed):
    idx0 = idx[:, :, 0].reshape(_N)
    out = _gather(node_type_embed, idx0)
    return out.reshape(_B, _L, _D)
